# Initial kernel scaffold; baseline (speedup 1.0000x reference)
import jax
import jax.numpy as jnp
from jax.experimental import pallas as pl


def kernel(input, weights):
    raise NotImplementedError("write your pallas kernel here")


# speedup vs baseline: 2.2756x; 2.2756x over previous
import functools
import jax
import jax.numpy as jnp
from jax import lax
from jax.experimental import pallas as pl
from jax.experimental.pallas import tpu as pltpu
from jax.experimental.pallas import tpu_sc as plsc

_OFFSET = 2
_PAD = 1

_info = plsc.get_sparse_core_info()
_NC, _NS = _info.num_cores, _info.num_subcores
_NW = _NC * _NS


def _pos_body(inp_ref, pos_ref):
    x = inp_ref[...]
    mask = (x != _PAD).astype(jnp.int32)
    c = mask
    k = 1
    n = x.shape[1]
    zrow = jnp.zeros_like(c)
    while k < n:
        shifted = jnp.concatenate([zrow[:, :k], c[:, :-k]], axis=1)
        c = c + shifted
        k *= 2
    pos_ref[...] = c * mask + (_OFFSET - 1)


def _positions(inp):
    return pl.pallas_call(
        _pos_body,
        out_shape=jax.ShapeDtypeStruct(inp.shape, jnp.int32),
    )(inp)


@functools.lru_cache(maxsize=None)
def _make_gather(N, D, CB):
    n_per_w = N // _NW
    nchunk = n_per_w // CB
    mesh = plsc.VectorSubcoreMesh(core_axis_name="c", subcore_axis_name="s")

    @functools.partial(
        pl.kernel,
        mesh=mesh,
        out_type=jax.ShapeDtypeStruct((N, D), jnp.float32),
        scratch_types=[
            pltpu.VMEM((nchunk, CB), jnp.int32),
            pltpu.VMEM((CB, D), jnp.float32),
            pltpu.VMEM((CB, D), jnp.float32),
            pltpu.SemaphoreType.DMA,
            pltpu.SemaphoreType.DMA,
            pltpu.SemaphoreType.DMA,
            pltpu.SemaphoreType.DMA,
        ],
    )
    def gather(pos_hbm, tab_hbm, out_hbm, idx_v, buf0, buf1, g0, g1, o0, o1):
        wid = lax.axis_index("s") * _NC + lax.axis_index("c")
        base = wid * n_per_w
        pltpu.sync_copy(pos_hbm.at[wid], idx_v)
        bufs = (buf0, buf1)
        gsem = (g0, g1)
        osem = (o0, o1)
        pltpu.async_copy(tab_hbm.at[idx_v.at[0]], buf0, g0)
        for c in range(nchunk):
            b = c % 2
            if c + 1 < nchunk:
                if c >= 1:
                    pltpu.make_async_copy(
                        bufs[1 - b],
                        out_hbm.at[pl.ds(base + (c - 1) * CB, CB)],
                        osem[1 - b],
                    ).wait()
                pltpu.async_copy(
                    tab_hbm.at[idx_v.at[c + 1]], bufs[1 - b], gsem[1 - b]
                )
            pltpu.make_async_copy(
                tab_hbm.at[idx_v.at[c]], bufs[b], gsem[b]
            ).wait()
            pltpu.async_copy(
                bufs[b], out_hbm.at[pl.ds(base + c * CB, CB)], osem[b]
            )
        for c in (nchunk - 2, nchunk - 1):
            b = c % 2
            pltpu.make_async_copy(
                bufs[b], out_hbm.at[pl.ds(base + c * CB, CB)], osem[b]
            ).wait()

    return gather


def kernel(input, weights):
    bsz, seq_len = input.shape
    N = bsz * seq_len
    D = weights.shape[1]
    CB = 32
    positions = _positions(input)
    pos3 = positions.reshape(_NW, N // (_NW * CB), CB)
    out = _make_gather(N, D, CB)(pos3, weights)
    return out.reshape(bsz, seq_len, D)
